# baseline (speedup 1.0000x reference)
import functools

import jax
import jax.numpy as jnp
from jax import lax
from jax.experimental import pallas as pl
from jax.experimental.pallas import tpu as pltpu
from jax.experimental.pallas import tpu_sc as plsc

EMB_D = 64
PAD_D = 128
_C = 64
_OC = 40
_NSUB = 16


def _ceil_to(x, m):
    return ((x + m - 1) // m) * m


_DCH = 640


@functools.cache
def _sc_segsum(n_links_pad, n_dst, n_parts):
    q = _ceil_to(-(-n_dst // n_parts), 160)
    spr = _ceil_to(q + 1, 1024)
    nt = n_links_pad // _NSUB
    jg = nt // _C
    zd = spr // _NSUB // 64
    och = q // _OC
    ojmax = -(-och // _NSUB)

    mesh = plsc.VectorSubcoreMesh(core_axis_name="c", subcore_axis_name="s")

    def body(table, sidx, didx, out, acc, sidx_v, didx_c, adj2,
             rows, sem_i, sem_z, sem_g, sem_s, sem_o):
        c = lax.axis_index("c")
        t = lax.axis_index("s")

        cp_s = pltpu.async_copy(sidx.at[pl.ds(t * nt, nt)], sidx_v, sem_i)

        def fire_gather(j, b):
            pltpu.async_copy(
                table.at[sidx_v.at[pl.ds(j * _C, _C)]], rows.at[b], sem_g)

        def wait_gather():
            pltpu.make_async_copy(
                table.at[sidx_v.at[pl.ds(0, _C)]], rows.at[0], sem_g).wait()

        def fire_scatter(j, b):
            pltpu.async_copy(rows.at[b], acc.at[adj2.at[j]], sem_s,
                             add=True)

        def wait_scatter():
            pltpu.make_async_copy(rows.at[0], acc.at[adj2.at[0]],
                                  sem_s).wait()

        zr = spr // _NSUB
        zvec = jnp.zeros((16,), jnp.float32)
        for p in range(n_parts // 2):
            for i in range(_C):
                for g in range(PAD_D // 16):
                    rows[0, i, pl.ds(16 * g, 16)] = zvec
            for j in range(zd):
                pltpu.async_copy(rows.at[0],
                                 acc.at[pl.ds(t * zr + _C * j, _C)], sem_z)

            off = (p * 2 + c) * q

            def adj_body(s, carry):
                pltpu.sync_copy(didx.at[pl.ds(t * nt + s * _DCH, _DCH)],
                                didx_c)
                for r in range(_DCH // _C):
                    for g in range(_C // 16):
                        w = didx_c[pl.ds(r * _C + 16 * g, 16)]
                        loc = w - off
                        ok = (loc >= 0) & (loc < q)
                        adj2[s * (_DCH // _C) + r, pl.ds(16 * g, 16)] = (
                            jnp.where(ok, loc, q))
                return carry

            lax.fori_loop(0, nt // _DCH, adj_body, 0)
            if p == 0:
                cp_s.wait()
            for j in range(zd):
                pltpu.make_async_copy(rows.at[0], acc.at[pl.ds(0, _C)],
                                      sem_z).wait()
            plsc.subcore_barrier()

            fire_gather(0, 0)
            fire_gather(1, 1)

            def pipe_body(j, carry):
                wait_gather()
                fire_scatter(j - 2, (j - 2) % 3)

                @pl.when(j >= 3)
                def _():
                    wait_scatter()
                fire_gather(j, j % 3)
                return carry

            lax.fori_loop(2, jg, pipe_body, 0)
            wait_gather()
            fire_scatter(jg - 2, (jg - 2) % 3)
            wait_gather()
            fire_scatter(jg - 1, (jg - 1) % 3)
            for _ in range(3):
                wait_scatter()
            plsc.subcore_barrier()

            def out_body(j, carry):
                oc = t + _NSUB * j

                @pl.when(oc < och)
                def _():
                    r0 = oc * _OC
                    pltpu.async_copy(acc.at[pl.ds(r0, _OC)],
                                     out.at[pl.ds(off + r0, _OC)], sem_o)

                @pl.when((j >= 2) & (t + _NSUB * (j - 2) < och))
                def _():
                    pltpu.make_async_copy(
                        acc.at[pl.ds(0, _OC)], out.at[pl.ds(0, _OC)],
                        sem_o).wait()
                return carry

            lax.fori_loop(0, ojmax + 2, out_body, 0)
            plsc.subcore_barrier()

    return pl.kernel(
        body,
        out_type=jax.ShapeDtypeStruct((n_parts * q, PAD_D), jnp.float32),
        mesh=mesh,
        scratch_types=[
            pltpu.VMEM_SHARED((spr, PAD_D), jnp.float32),
            pltpu.VMEM((nt,), jnp.int32),
            pltpu.VMEM((_DCH,), jnp.int32),
            pltpu.VMEM((jg, _C), jnp.int32),
            pltpu.VMEM((3, _C, PAD_D), jnp.float32),
            pltpu.SemaphoreType.DMA,
            pltpu.SemaphoreType.DMA,
            pltpu.SemaphoreType.DMA,
            pltpu.SemaphoreType.DMA,
            pltpu.SemaphoreType.DMA,
        ],
        compiler_params=pltpu.CompilerParams(use_tc_tiling_on_sc=True),
        name=f"sc_segsum_{n_links_pad}_{n_dst}_{n_parts}",
    )


_BN = 2000


@functools.cache
def _tc_proj(n_rows, n_feat):
    def body(x_ref, w_ref, b_ref, o_ref):
        r = jnp.maximum(
            jnp.dot(x_ref[...], w_ref[...],
                    preferred_element_type=jnp.float32) + b_ref[...], 0.0)
        o_ref[...] = jnp.concatenate(
            [r, jnp.zeros((_BN, PAD_D - EMB_D), jnp.float32)], axis=1)

    return pl.pallas_call(
        body,
        grid=(n_rows // _BN,),
        in_specs=[
            pl.BlockSpec((_BN, n_feat), lambda i: (i, 0)),
            pl.BlockSpec((n_feat, EMB_D), lambda i: (0, 0)),
            pl.BlockSpec((1, EMB_D), lambda i: (0, 0)),
        ],
        out_specs=pl.BlockSpec((_BN, PAD_D), lambda i: (i, 0)),
        out_shape=jax.ShapeDtypeStruct((n_rows, PAD_D), jnp.float32),
        name=f"tc_proj_{n_rows}_{n_feat}",
    )


@functools.cache
def _tc_update(n_rows):
    def body(d_ref, a_ref, w_ref, o_ref):
        r = jnp.maximum(
            d_ref[:, :EMB_D] + jnp.dot(a_ref[...], w_ref[...],
                                       preferred_element_type=jnp.float32),
            0.0)
        o_ref[...] = jnp.concatenate(
            [r, jnp.zeros((_BN, PAD_D - EMB_D), jnp.float32)], axis=1)

    return pl.pallas_call(
        body,
        grid=(n_rows // _BN,),
        in_specs=[
            pl.BlockSpec((_BN, PAD_D), lambda i: (i, 0)),
            pl.BlockSpec((_BN, PAD_D), lambda i: (i, 0)),
            pl.BlockSpec((PAD_D, EMB_D), lambda i: (0, 0)),
        ],
        out_specs=pl.BlockSpec((_BN, PAD_D), lambda i: (i, 0)),
        out_shape=jax.ShapeDtypeStruct((n_rows, PAD_D), jnp.float32),
        name=f"tc_update_{n_rows}",
    )


def kernel(left_faces, left_loops, left_edges, left_verts,
           right_faces, right_loops, right_edges, right_verts,
           left_face_to_loop, left_loop_to_edge, left_edge_to_vertex,
           left_face_to_face,
           right_face_to_loop, right_loop_to_edge, right_edge_to_vertex,
           right_face_to_face,
           Wf, bf, Wl, bl, We, be, Wv, bv,
           W_ve, W_el, W_lf, W_ff, W_fl, W_le, W_ev):
    K = 6
    b2 = lambda b: b.reshape(1, EMB_D)
    wp = lambda w: jnp.pad(w, ((0, PAD_D - EMB_D), (0, 0)))
    Wp_ve, Wp_el, Wp_lf, Wp_ff, Wp_fl, Wp_le, Wp_ev = (
        wp(W_ve), wp(W_el), wp(W_lf), wp(W_ff), wp(W_fl), wp(W_le), wp(W_ev))

    def side(faces, loops, edges, verts, f2l, l2e, e2v, f2f):
        F_N, L_N, E_N, V_N = (faces.shape[0], loops.shape[0],
                              edges.shape[0], verts.shape[0])
        f = _tc_proj(F_N, faces.shape[1])(faces, Wf, b2(bf))
        l = _tc_proj(L_N, loops.shape[1])(loops, Wl, b2(bl))
        e = _tc_proj(E_N, edges.shape[1])(edges, We, b2(be))
        v = _tc_proj(V_N, verts.shape[1])(verts, Wv, b2(bv))

        def prep(s_idx, d_idx, n_dst):
            n = s_idx.shape[0]
            nl = _ceil_to(n, _NSUB * _DCH)
            si = jnp.concatenate([s_idx, jnp.zeros((nl - n,), jnp.int32)])
            di = jnp.concatenate(
                [d_idx, jnp.full((nl - n,), n_dst, jnp.int32)])
            return si, di, nl

        def hop(src, dst, s_idx, d_idx, n_dst, Wp):
            si, di, nl = prep(s_idx, d_idx, n_dst)
            n_parts = 4 if n_dst > 20000 else 2
            agg = _sc_segsum(nl, n_dst, n_parts)(src, si, di)
            return _tc_update(n_dst)(dst, agg, Wp)

        for _ in range(K):
            e = hop(v, e, e2v[1], e2v[0], E_N, Wp_ve)
            l = hop(e, l, l2e[1], l2e[0], L_N, Wp_el)
            f = hop(l, f, f2l[1], f2l[0], F_N, Wp_lf)
            f = hop(f, f, f2f[1], f2f[0], F_N, Wp_ff)
            l = hop(f, l, f2l[0], f2l[1], L_N, Wp_fl)
            e = hop(l, e, l2e[0], l2e[1], E_N, Wp_le)
            v = hop(e, v, e2v[0], e2v[1], V_N, Wp_ev)
        return f[:, :EMB_D], e[:, :EMB_D], v[:, :EMB_D]

    out_l = side(left_faces, left_loops, left_edges, left_verts,
                 left_face_to_loop, left_loop_to_edge, left_edge_to_vertex,
                 left_face_to_face)
    out_r = side(right_faces, right_loops, right_edges, right_verts,
                 right_face_to_loop, right_loop_to_edge, right_edge_to_vertex,
                 right_face_to_face)
    return (out_l, out_r)
